# baseline (speedup 1.0000x reference)
import functools

import jax
import jax.numpy as jnp
from jax import lax
from jax.experimental import pallas as pl
from jax.experimental.pallas import tpu as pltpu
from jax.experimental.pallas import tpu_sc as plsc

_E = 16
_TOP_K = 2
_ALPHA = 0.01
_LANES = 16
_NWORK = 32


def _logits_kernel(x_ref, w_ref, out_ref):
    out_ref[...] = jax.lax.dot_general(
        w_ref[...], x_ref[...],
        dimension_numbers=(((1,), (1,)), ((), ())),
        preferred_element_type=jnp.float32,
    )


def _sc_gate(logits_hbm, i1_hbm, i2_hbm, w1_hbm, w2_hbm, r1_hbm, r2_hbm,
             pi_hbm, cnt_hbm,
             log_v, i1_v, i2_v, w1_v, w2_v, r1_v, r2_v, pi_v, cnt_v,
             *, n_tokens):
    tpw = n_tokens // _NWORK
    nb = tpw // _LANES
    wid = lax.axis_index("s") * 2 + lax.axis_index("c")
    base = wid * tpw

    pltpu.sync_copy(logits_hbm.at[:, pl.ds(base, tpw)], log_v)

    for e in range(_E):
        pi_v[e] = jnp.zeros((_LANES,), jnp.float32)
        cnt_v[e] = jnp.zeros((_LANES,), jnp.float32)

    lane = lax.iota(jnp.int32, _LANES)

    def batch(j, carry):
        off = j * _LANES
        s = [log_v[e, pl.ds(off, _LANES)] for e in range(_E)]
        m = s[0]
        for e in range(1, _E):
            m = jnp.maximum(m, s[e])
        ex = [jnp.exp(s[e] - m) for e in range(_E)]
        tot = ex[0]
        for e in range(1, _E):
            tot = tot + ex[e]
        inv = 1.0 / tot
        p = [ex[e] * inv for e in range(_E)]

        m1 = p[0]
        for e in range(1, _E):
            m1 = jnp.maximum(m1, p[e])
        i1 = jnp.where(p[0] == m1, 0, _E)
        for e in range(1, _E):
            i1 = jnp.minimum(i1, jnp.where(p[e] == m1, e, _E))

        pm = [jnp.where(i1 == e, -jnp.inf, p[e]) for e in range(_E)]
        m2 = pm[0]
        for e in range(1, _E):
            m2 = jnp.maximum(m2, pm[e])
        i2 = jnp.where(pm[0] == m2, 0, _E)
        for e in range(1, _E):
            i2 = jnp.minimum(i2, jnp.where(pm[e] == m2, e, _E))

        i1_v[pl.ds(off, _LANES)] = i1
        i2_v[pl.ds(off, _LANES)] = i2
        w1_v[pl.ds(off, _LANES)] = m1
        w2_v[pl.ds(off, _LANES)] = m2
        tok = lane + (base + off)
        r1_v[pl.ds(off, _LANES)] = tok
        r2_v[pl.ds(off, _LANES)] = tok + n_tokens

        for e in range(_E):
            plsc.addupdate(pi_v.at[e], p[e])
            c = (jnp.where(i1 == e, 1.0, 0.0)
                 + jnp.where(i2 == e, 1.0, 0.0))
            plsc.addupdate(cnt_v.at[e], c)
        return carry

    lax.fori_loop(0, nb, batch, 0)

    pltpu.sync_copy(i1_v, i1_hbm.at[pl.ds(base, tpw)])
    pltpu.sync_copy(i2_v, i2_hbm.at[pl.ds(base, tpw)])
    pltpu.sync_copy(w1_v, w1_hbm.at[pl.ds(base, tpw)])
    pltpu.sync_copy(w2_v, w2_hbm.at[pl.ds(base, tpw)])
    pltpu.sync_copy(r1_v, r1_hbm.at[pl.ds(base, tpw)])
    pltpu.sync_copy(r2_v, r2_hbm.at[pl.ds(base, tpw)])
    pltpu.sync_copy(pi_v, pi_hbm.at[wid])
    pltpu.sync_copy(cnt_v, cnt_hbm.at[wid])


def _aux_kernel(pi_ref, cnt_ref, aux_ref, *, scale):
    pi = jnp.sum(jnp.sum(pi_ref[...], axis=0), axis=1,
                 keepdims=True)
    cnt = jnp.sum(jnp.sum(cnt_ref[...], axis=0), axis=1,
                  keepdims=True)
    aux_ref[...] = jnp.sum(pi * cnt, axis=(0, 1), keepdims=True) * scale


def kernel(hidden_states, weight):
    bsz, seq_len, h = hidden_states.shape
    n = bsz * seq_len
    x = hidden_states.reshape(n, h)
    blk = 1024
    tpw = n // _NWORK

    logits_t = pl.pallas_call(
        _logits_kernel,
        grid=(n // blk,),
        in_specs=[
            pl.BlockSpec((blk, h), lambda i: (i, 0)),
            pl.BlockSpec((_E, h), lambda i: (0, 0)),
        ],
        out_specs=pl.BlockSpec((_E, blk), lambda i: (0, i)),
        out_shape=jax.ShapeDtypeStruct((_E, n), jnp.float32),
    )(x, weight)

    mesh = plsc.VectorSubcoreMesh(core_axis_name="c", subcore_axis_name="s")
    i1, i2, w1, w2, r1, r2, pi_p, cnt_p = pl.kernel(
        functools.partial(_sc_gate, n_tokens=n),
        out_type=[
            jax.ShapeDtypeStruct((n,), jnp.int32),
            jax.ShapeDtypeStruct((n,), jnp.int32),
            jax.ShapeDtypeStruct((n,), jnp.float32),
            jax.ShapeDtypeStruct((n,), jnp.float32),
            jax.ShapeDtypeStruct((n,), jnp.int32),
            jax.ShapeDtypeStruct((n,), jnp.int32),
            jax.ShapeDtypeStruct((_NWORK, _E, _LANES), jnp.float32),
            jax.ShapeDtypeStruct((_NWORK, _E, _LANES), jnp.float32),
        ],
        mesh=mesh,
        scratch_types=[
            pltpu.VMEM((_E, tpw), jnp.float32),
            pltpu.VMEM((tpw,), jnp.int32),
            pltpu.VMEM((tpw,), jnp.int32),
            pltpu.VMEM((tpw,), jnp.float32),
            pltpu.VMEM((tpw,), jnp.float32),
            pltpu.VMEM((tpw,), jnp.int32),
            pltpu.VMEM((tpw,), jnp.int32),
            pltpu.VMEM((_E, _LANES), jnp.float32),
            pltpu.VMEM((_E, _LANES), jnp.float32),
        ],
    )(logits_t)

    scale = (_E * _ALPHA) / (float(n) * float(n) * _TOP_K)
    aux = pl.pallas_call(
        functools.partial(_aux_kernel, scale=scale),
        out_shape=jax.ShapeDtypeStruct((1, 1), jnp.float32),
    )(pi_p, cnt_p)

    topk_idx = jnp.stack([i1, i2], axis=1)
    topk_weight = jnp.stack([w1, w2], axis=1)
    row_idx = jnp.stack([r1, r2], axis=1)
    return (topk_idx, topk_weight, row_idx, aux[0, 0])


# speedup vs baseline: 1.6195x; 1.6195x over previous
import functools

import jax
import jax.numpy as jnp
from jax.experimental import pallas as pl
from jax.experimental.pallas import tpu as pltpu

_E = 16
_TOP_K = 2
_ALPHA = 0.01


def _gate_kernel(x_ref, w_ref, idx_ref, wgt_ref, row_ref, aux_ref,
                 acc_ref, *, blk, n_tokens):
    step = pl.program_id(0)
    nsteps = pl.num_programs(0)

    logits = jax.lax.dot_general(
        w_ref[...], x_ref[...],
        dimension_numbers=(((1,), (1,)), ((), ())),
        preferred_element_type=jnp.float32,
    )

    m = jnp.max(logits, axis=0, keepdims=True)
    e = jnp.exp(logits - m)
    s = jnp.sum(e, axis=0, keepdims=True)
    scores = e / s

    expert_iota = jax.lax.broadcasted_iota(jnp.int32, (_E, blk), 0)

    m1 = jnp.max(scores, axis=0, keepdims=True)
    i1 = jnp.min(jnp.where(scores == m1, expert_iota, _E),
                 axis=0, keepdims=True)
    masked = jnp.where(expert_iota == i1, -jnp.inf, scores)
    m2 = jnp.max(masked, axis=0, keepdims=True)
    i2 = jnp.min(jnp.where(masked == m2, expert_iota, _E),
                 axis=0, keepdims=True)

    base = step * blk
    idx_ref[0:1, pl.ds(base, blk)] = i1
    idx_ref[1:2, pl.ds(base, blk)] = i2
    wgt_ref[0:1, pl.ds(base, blk)] = m1
    wgt_ref[1:2, pl.ds(base, blk)] = m2

    tok = jax.lax.broadcasted_iota(jnp.int32, (1, blk), 1) + base
    row_ref[0:1, pl.ds(base, blk)] = tok
    row_ref[1:2, pl.ds(base, blk)] = tok + n_tokens

    score_sum = jnp.sum(scores, axis=1, keepdims=True)
    cnt = (jnp.sum((expert_iota == i1).astype(jnp.float32), axis=1,
                   keepdims=True)
           + jnp.sum((expert_iota == i2).astype(jnp.float32), axis=1,
                     keepdims=True))

    @pl.when(step == 0)
    def _init():
        acc_ref[...] = jnp.zeros_like(acc_ref)

    acc_ref[:, 0:1] += score_sum
    acc_ref[:, 1:2] += cnt

    @pl.when(step == nsteps - 1)
    def _finalize():
        pi = acc_ref[:, 0:1] / n_tokens
        ce = acc_ref[:, 1:2] / (n_tokens * _TOP_K)
        aux_ref[...] = jnp.sum(pi * ce, axis=(0, 1),
                               keepdims=True) * (_E * _ALPHA)


def kernel(hidden_states, weight):
    bsz, seq_len, h = hidden_states.shape
    n = bsz * seq_len
    x = hidden_states.reshape(n, h)

    blk = 1024
    grid = n // blk

    idx_t, wgt_t, row_t, aux = pl.pallas_call(
        functools.partial(_gate_kernel, blk=blk, n_tokens=n),
        grid=(grid,),
        in_specs=[
            pl.BlockSpec((blk, h), lambda i: (i, 0)),
            pl.BlockSpec((_E, h), lambda i: (0, 0)),
        ],
        out_specs=[
            pl.BlockSpec((_TOP_K, n), lambda i: (0, 0)),
            pl.BlockSpec((_TOP_K, n), lambda i: (0, 0)),
            pl.BlockSpec((_TOP_K, n), lambda i: (0, 0)),
            pl.BlockSpec((1, 1), lambda i: (0, 0)),
        ],
        out_shape=[
            jax.ShapeDtypeStruct((_TOP_K, n), jnp.int32),
            jax.ShapeDtypeStruct((_TOP_K, n), jnp.float32),
            jax.ShapeDtypeStruct((_TOP_K, n), jnp.int32),
            jax.ShapeDtypeStruct((1, 1), jnp.float32),
        ],
        scratch_shapes=[pltpu.VMEM((_E, 2), jnp.float32)],
    )(x, weight)

    return (idx_t.T, wgt_t.T, row_t.T, aux[0, 0])
